# Initial kernel scaffold; baseline (speedup 1.0000x reference)
import jax
import jax.numpy as jnp
from jax.experimental import pallas as pl


def kernel(x, edge_index, W1, b1, W2, b2, W_age, b_age, W_sex, b_sex, W_eth, b_eth):
    raise NotImplementedError("write your pallas kernel here")


# speedup vs baseline: 14.0824x; 14.0824x over previous
import functools

import jax
import jax.numpy as jnp
from jax import lax
from jax.experimental import pallas as pl
from jax.experimental.pallas import tpu as pltpu
from jax.experimental.pallas import tpu_sc as plsc

N = 10000
E = 320000
D = 128
NC = 2
NS = 16
NW = NC * NS
ROWS_PAD = 10240
RPS = ROWS_PAD // NS
EPT = E // NW
CH = 80
NCHUNK = EPT // CH
BR = 2000


def _sc_mesh():
    return plsc.VectorSubcoreMesh(core_axis_name="c", subcore_axis_name="s")


def _sc_degree(dst, zeros, ones):
    @functools.partial(
        pl.kernel,
        mesh=_sc_mesh(),
        out_type=jax.ShapeDtypeStruct((NC, ROWS_PAD, D), jnp.float32),
        scratch_types=[
            pltpu.VMEM((CH,), jnp.int32),
            pltpu.VMEM((CH, D), jnp.float32),
            pltpu.VMEM_SHARED((ROWS_PAD, D), jnp.float32),
        ],
    )
    def k(dst_hbm, z_hbm, ones_hbm, out_hbm, idx_v, ones_v, shared):
        cid = lax.axis_index("c")
        sid = lax.axis_index("s")
        rows0 = sid * RPS
        pltpu.sync_copy(z_hbm.at[pl.ds(rows0, RPS)], shared.at[pl.ds(rows0, RPS)])
        pltpu.sync_copy(ones_hbm, ones_v)
        plsc.subcore_barrier()
        base = (cid * NS + sid) * EPT

        @pl.loop(0, NCHUNK)
        def _(c):
            pltpu.sync_copy(dst_hbm.at[pl.ds(base + c * CH, CH)], idx_v)
            pltpu.sync_copy(ones_v, shared.at[idx_v], add=True)

        plsc.subcore_barrier()
        pltpu.sync_copy(shared.at[pl.ds(rows0, RPS)],
                        out_hbm.at[cid, pl.ds(rows0, RPS)])

    return k(dst, zeros, ones)


def _sc_aggregate(y, src, dst, zeros):
    @functools.partial(
        pl.kernel,
        mesh=_sc_mesh(),
        out_type=jax.ShapeDtypeStruct((NC, ROWS_PAD, D), jnp.float32),
        scratch_types=[
            pltpu.VMEM((CH,), jnp.int32),
            pltpu.VMEM((CH,), jnp.int32),
            pltpu.VMEM((CH, D), jnp.float32),
            pltpu.VMEM_SHARED((ROWS_PAD, D), jnp.float32),
        ],
    )
    def k(y_hbm, src_hbm, dst_hbm, z_hbm, out_hbm, si_v, di_v, rows_v, shared):
        cid = lax.axis_index("c")
        sid = lax.axis_index("s")
        rows0 = sid * RPS
        pltpu.sync_copy(z_hbm.at[pl.ds(rows0, RPS)], shared.at[pl.ds(rows0, RPS)])
        plsc.subcore_barrier()
        base = (cid * NS + sid) * EPT

        @pl.loop(0, NCHUNK)
        def _(c):
            pltpu.sync_copy(src_hbm.at[pl.ds(base + c * CH, CH)], si_v)
            pltpu.sync_copy(dst_hbm.at[pl.ds(base + c * CH, CH)], di_v)
            pltpu.sync_copy(y_hbm.at[si_v], rows_v)
            pltpu.sync_copy(rows_v, shared.at[di_v], add=True)

        plsc.subcore_barrier()
        pltpu.sync_copy(shared.at[pl.ds(rows0, RPS)],
                        out_hbm.at[cid, pl.ds(rows0, RPS)])

    return k(y, src, dst, zeros)


def _tc_matmul(x, W):
    def body(x_ref, w_ref, o_ref):
        o_ref[...] = jnp.dot(x_ref[...], w_ref[...],
                             preferred_element_type=jnp.float32,
                             precision=lax.Precision.HIGHEST)

    return pl.pallas_call(
        body,
        grid=(N // BR,),
        in_specs=[pl.BlockSpec((BR, D), lambda i: (i, 0)),
                  pl.BlockSpec((D, D), lambda i: (0, 0))],
        out_specs=pl.BlockSpec((BR, D), lambda i: (i, 0)),
        out_shape=jax.ShapeDtypeStruct((N, D), jnp.float32),
    )(x, W)


def _tc_scale_from_deg(degp, xw):
    def body(degp_ref, xw_ref, y_ref, dis_ref):
        deg = degp_ref[0, :, 0:1] + degp_ref[1, :, 0:1] + 1.0
        dis = lax.rsqrt(deg)
        y_ref[...] = xw_ref[...] * dis
        dis_ref[...] = jnp.broadcast_to(dis, (BR, D))

    return pl.pallas_call(
        body,
        grid=(N // BR,),
        in_specs=[pl.BlockSpec((NC, BR, D), lambda i: (0, i, 0)),
                  pl.BlockSpec((BR, D), lambda i: (i, 0))],
        out_specs=[pl.BlockSpec((BR, D), lambda i: (i, 0)),
                   pl.BlockSpec((BR, D), lambda i: (i, 0))],
        out_shape=[jax.ShapeDtypeStruct((N, D), jnp.float32),
                   jax.ShapeDtypeStruct((N, D), jnp.float32)],
    )(degp, xw)


def _tc_combine_matmul(p, y, dis, W, b):
    def body(p_ref, y_ref, dis_ref, w_ref, b_ref, o_ref):
        s = p_ref[0] + p_ref[1] + y_ref[...]
        h = jnp.maximum(dis_ref[...] * s + b_ref[...], 0.0)
        o_ref[...] = dis_ref[...] * jnp.dot(h, w_ref[...],
                                            preferred_element_type=jnp.float32,
                                            precision=lax.Precision.HIGHEST)

    return pl.pallas_call(
        body,
        grid=(N // BR,),
        in_specs=[pl.BlockSpec((NC, BR, D), lambda i: (0, i, 0)),
                  pl.BlockSpec((BR, D), lambda i: (i, 0)),
                  pl.BlockSpec((BR, D), lambda i: (i, 0)),
                  pl.BlockSpec((D, D), lambda i: (0, 0)),
                  pl.BlockSpec((1, D), lambda i: (0, 0))],
        out_specs=pl.BlockSpec((BR, D), lambda i: (i, 0)),
        out_shape=jax.ShapeDtypeStruct((N, D), jnp.float32),
    )(p, y, dis, W, b)


def _tc_combine_scale(p, y, dis, b):
    def body(p_ref, y_ref, dis_ref, b_ref, o_ref):
        s = p_ref[0] + p_ref[1] + y_ref[...]
        h = jnp.maximum(dis_ref[...] * s + b_ref[...], 0.0)
        o_ref[...] = dis_ref[...] * h

    return pl.pallas_call(
        body,
        grid=(N // BR,),
        in_specs=[pl.BlockSpec((NC, BR, D), lambda i: (0, i, 0)),
                  pl.BlockSpec((BR, D), lambda i: (i, 0)),
                  pl.BlockSpec((BR, D), lambda i: (i, 0)),
                  pl.BlockSpec((1, D), lambda i: (0, 0))],
        out_specs=pl.BlockSpec((BR, D), lambda i: (i, 0)),
        out_shape=jax.ShapeDtypeStruct((N, D), jnp.float32),
    )(p, y, dis, b)


def _tc_heads(p, y, dis, Wh, bh):
    DH = Wh.shape[1]

    def body(p_ref, y_ref, dis_ref, w_ref, b_ref, o_ref):
        agg = dis_ref[...] * (p_ref[0] + p_ref[1] + y_ref[...])
        o_ref[...] = jnp.dot(agg, w_ref[...],
                             preferred_element_type=jnp.float32,
                             precision=lax.Precision.HIGHEST) + b_ref[...]

    return pl.pallas_call(
        body,
        grid=(N // BR,),
        in_specs=[pl.BlockSpec((NC, BR, D), lambda i: (0, i, 0)),
                  pl.BlockSpec((BR, D), lambda i: (i, 0)),
                  pl.BlockSpec((BR, D), lambda i: (i, 0)),
                  pl.BlockSpec((D, DH), lambda i: (0, 0)),
                  pl.BlockSpec((1, DH), lambda i: (0, 0))],
        out_specs=pl.BlockSpec((BR, DH), lambda i: (i, 0)),
        out_shape=jax.ShapeDtypeStruct((N, DH), jnp.float32),
    )(p, y, dis, Wh, bh)


def kernel(x, edge_index, W1, b1, W2, b2, W_age, b_age, W_sex, b_sex, W_eth, b_eth):
    src = edge_index[0]
    dst = edge_index[1]
    zeros128 = jnp.zeros((ROWS_PAD, D), jnp.float32)
    ones128 = jnp.ones((CH, D), jnp.float32)
    Wh = jnp.concatenate([W_age, W_sex, W_eth], axis=1)
    bh = jnp.concatenate([b_age, b_sex, b_eth]).reshape(1, -1)

    degp = _sc_degree(dst, zeros128, ones128)
    xw1 = _tc_matmul(x, W1)
    y1, dis = _tc_scale_from_deg(degp, xw1)

    p1 = _sc_aggregate(y1, src, dst, zeros128)
    y2 = _tc_combine_matmul(p1, y1, dis, W2, b1.reshape(1, -1))

    p2 = _sc_aggregate(y2, src, dst, zeros128)
    y3 = _tc_combine_scale(p2, y2, dis, b2.reshape(1, -1))

    p3 = _sc_aggregate(y3, src, dst, zeros128)
    heads = _tc_heads(p3, y3, dis, Wh, bh)

    return (heads[:, 0:3], heads[:, 3:5], heads[:, 5:8])


# speedup vs baseline: 15.3066x; 1.0869x over previous
import functools

import jax
import jax.numpy as jnp
from jax import lax
from jax.experimental import pallas as pl
from jax.experimental.pallas import tpu as pltpu
from jax.experimental.pallas import tpu_sc as plsc

N = 10000
E = 320000
D = 128
NC = 2
NS = 16
NW = NC * NS
ROWS_PAD = 10240
RPS = ROWS_PAD // NS
CH = 128
NCHUNK = 79
E_PAD = NW * NCHUNK * CH
TRASH = ROWS_PAD - 1
BR = 2000


def _sc_mesh():
    return plsc.VectorSubcoreMesh(core_axis_name="c", subcore_axis_name="s")


def _sc_degree(dst, zeros, ones):
    @functools.partial(
        pl.kernel,
        mesh=_sc_mesh(),
        out_type=jax.ShapeDtypeStruct((NC, ROWS_PAD, D), jnp.float32),
        scratch_types=[
            pltpu.VMEM((NCHUNK, CH), jnp.int32),
            pltpu.VMEM((CH, D), jnp.float32),
            pltpu.VMEM_SHARED((ROWS_PAD, D), jnp.float32),
            pltpu.SemaphoreType.DMA,
            pltpu.SemaphoreType.DMA,
        ],
    )
    def k(dst_hbm, z_hbm, ones_hbm, out_hbm, di, ones_v, shared, s0, s1):
        cid = lax.axis_index("c")
        sid = lax.axis_index("s")
        wid = cid * NS + sid
        rows0 = sid * RPS
        pltpu.sync_copy(z_hbm.at[pl.ds(rows0, RPS)], shared.at[pl.ds(rows0, RPS)])
        pltpu.sync_copy(dst_hbm.at[wid], di)
        pltpu.sync_copy(ones_hbm, ones_v)
        plsc.subcore_barrier()

        @pl.loop(0, NCHUNK, step=2)
        def _(c):
            pltpu.async_copy(ones_v, shared.at[di.at[c]], s0, add=True)

            @pl.when(c + 1 < NCHUNK)
            def _():
                pltpu.async_copy(ones_v, shared.at[di.at[c + 1]], s1, add=True)

            pltpu.make_async_copy(ones_v, shared.at[di.at[c]], s0).wait()

            @pl.when(c + 1 < NCHUNK)
            def _():
                pltpu.make_async_copy(ones_v, shared.at[di.at[c + 1]], s1).wait()

        plsc.subcore_barrier()
        pltpu.sync_copy(shared.at[pl.ds(rows0, RPS)],
                        out_hbm.at[cid, pl.ds(rows0, RPS)])

    return k(dst, zeros, ones)


def _sc_aggregate(y, idx, zeros):
    @functools.partial(
        pl.kernel,
        mesh=_sc_mesh(),
        out_type=jax.ShapeDtypeStruct((NC, ROWS_PAD, D), jnp.float32),
        scratch_types=[
            pltpu.VMEM((2, CH), jnp.int32),
            pltpu.VMEM((2, CH), jnp.int32),
            pltpu.VMEM((CH, D), jnp.float32),
            pltpu.VMEM((CH, D), jnp.float32),
            pltpu.VMEM_SHARED((ROWS_PAD, D), jnp.float32),
            pltpu.SemaphoreType.DMA,
            pltpu.SemaphoreType.DMA,
            pltpu.SemaphoreType.DMA,
            pltpu.SemaphoreType.DMA,
        ],
    )
    def k(y_hbm, idx_hbm, z_hbm, out_hbm,
          ix0, ix1, r0, r1, shared, i0, i1, g0, g1):
        cid = lax.axis_index("c")
        sid = lax.axis_index("s")
        wid = cid * NS + sid
        rows0 = sid * RPS
        pltpu.sync_copy(z_hbm.at[pl.ds(rows0, RPS)], shared.at[pl.ds(rows0, RPS)])
        pltpu.sync_copy(idx_hbm.at[wid, 0], ix0)
        plsc.subcore_barrier()
        pltpu.async_copy(idx_hbm.at[wid, 1], ix1, i1)
        pltpu.async_copy(y_hbm.at[ix0.at[0]], r0, g0)

        @pl.loop(0, NCHUNK, step=2)
        def _(c):
            @pl.when(c + 1 < NCHUNK)
            def _():
                pltpu.make_async_copy(idx_hbm.at[wid, c + 1], ix1, i1).wait()

            pltpu.make_async_copy(y_hbm.at[ix0.at[0]], r0, g0).wait()

            @pl.when(c + 1 < NCHUNK)
            def _():
                pltpu.async_copy(y_hbm.at[ix1.at[0]], r1, g1)

            pltpu.sync_copy(r0, shared.at[ix0.at[1]], add=True)

            @pl.when(c + 2 < NCHUNK)
            def _():
                pltpu.async_copy(idx_hbm.at[wid, c + 2], ix0, i0)

            @pl.when(c + 1 < NCHUNK)
            def _():
                pltpu.make_async_copy(y_hbm.at[ix1.at[0]], r1, g1).wait()

                @pl.when(c + 2 < NCHUNK)
                def _():
                    pltpu.make_async_copy(idx_hbm.at[wid, c + 2], ix0, i0).wait()
                    pltpu.async_copy(y_hbm.at[ix0.at[0]], r0, g0)

                pltpu.sync_copy(r1, shared.at[ix1.at[1]], add=True)

                @pl.when(c + 3 < NCHUNK)
                def _():
                    pltpu.async_copy(idx_hbm.at[wid, c + 3], ix1, i1)

        plsc.subcore_barrier()
        pltpu.sync_copy(shared.at[pl.ds(rows0, RPS)],
                        out_hbm.at[cid, pl.ds(rows0, RPS)])

    return k(y, idx, zeros)


def _tc_matmul(x, W):
    def body(x_ref, w_ref, o_ref):
        o_ref[...] = jnp.dot(x_ref[...], w_ref[...],
                             preferred_element_type=jnp.float32,
                             precision=lax.Precision.HIGHEST)

    return pl.pallas_call(
        body,
        grid=(N // BR,),
        in_specs=[pl.BlockSpec((BR, D), lambda i: (i, 0)),
                  pl.BlockSpec((D, D), lambda i: (0, 0))],
        out_specs=pl.BlockSpec((BR, D), lambda i: (i, 0)),
        out_shape=jax.ShapeDtypeStruct((N, D), jnp.float32),
    )(x, W)


def _tc_scale_from_deg(degp, xw):
    def body(degp_ref, xw_ref, y_ref, dis_ref):
        deg = degp_ref[0, :, 0:1] + degp_ref[1, :, 0:1] + 1.0
        dis = lax.rsqrt(deg)
        y_ref[...] = xw_ref[...] * dis
        dis_ref[...] = jnp.broadcast_to(dis, (BR, D))

    return pl.pallas_call(
        body,
        grid=(N // BR,),
        in_specs=[pl.BlockSpec((NC, BR, D), lambda i: (0, i, 0)),
                  pl.BlockSpec((BR, D), lambda i: (i, 0))],
        out_specs=[pl.BlockSpec((BR, D), lambda i: (i, 0)),
                   pl.BlockSpec((BR, D), lambda i: (i, 0))],
        out_shape=[jax.ShapeDtypeStruct((N, D), jnp.float32),
                   jax.ShapeDtypeStruct((N, D), jnp.float32)],
    )(degp, xw)


def _tc_combine_matmul(p, y, dis, W, b):
    def body(p_ref, y_ref, dis_ref, w_ref, b_ref, o_ref):
        s = p_ref[0] + p_ref[1] + y_ref[...]
        h = jnp.maximum(dis_ref[...] * s + b_ref[...], 0.0)
        o_ref[...] = dis_ref[...] * jnp.dot(h, w_ref[...],
                                            preferred_element_type=jnp.float32,
                                            precision=lax.Precision.HIGHEST)

    return pl.pallas_call(
        body,
        grid=(N // BR,),
        in_specs=[pl.BlockSpec((NC, BR, D), lambda i: (0, i, 0)),
                  pl.BlockSpec((BR, D), lambda i: (i, 0)),
                  pl.BlockSpec((BR, D), lambda i: (i, 0)),
                  pl.BlockSpec((D, D), lambda i: (0, 0)),
                  pl.BlockSpec((1, D), lambda i: (0, 0))],
        out_specs=pl.BlockSpec((BR, D), lambda i: (i, 0)),
        out_shape=jax.ShapeDtypeStruct((N, D), jnp.float32),
    )(p, y, dis, W, b)


def _tc_combine_scale(p, y, dis, b):
    def body(p_ref, y_ref, dis_ref, b_ref, o_ref):
        s = p_ref[0] + p_ref[1] + y_ref[...]
        h = jnp.maximum(dis_ref[...] * s + b_ref[...], 0.0)
        o_ref[...] = dis_ref[...] * h

    return pl.pallas_call(
        body,
        grid=(N // BR,),
        in_specs=[pl.BlockSpec((NC, BR, D), lambda i: (0, i, 0)),
                  pl.BlockSpec((BR, D), lambda i: (i, 0)),
                  pl.BlockSpec((BR, D), lambda i: (i, 0)),
                  pl.BlockSpec((1, D), lambda i: (0, 0))],
        out_specs=pl.BlockSpec((BR, D), lambda i: (i, 0)),
        out_shape=jax.ShapeDtypeStruct((N, D), jnp.float32),
    )(p, y, dis, b)


def _tc_heads(p, y, dis, Wh, bh):
    DH = Wh.shape[1]

    def body(p_ref, y_ref, dis_ref, w_ref, b_ref, o_ref):
        agg = dis_ref[...] * (p_ref[0] + p_ref[1] + y_ref[...])
        o_ref[...] = jnp.dot(agg, w_ref[...],
                             preferred_element_type=jnp.float32,
                             precision=lax.Precision.HIGHEST) + b_ref[...]

    return pl.pallas_call(
        body,
        grid=(N // BR,),
        in_specs=[pl.BlockSpec((NC, BR, D), lambda i: (0, i, 0)),
                  pl.BlockSpec((BR, D), lambda i: (i, 0)),
                  pl.BlockSpec((BR, D), lambda i: (i, 0)),
                  pl.BlockSpec((D, DH), lambda i: (0, 0)),
                  pl.BlockSpec((1, DH), lambda i: (0, 0))],
        out_specs=pl.BlockSpec((BR, DH), lambda i: (i, 0)),
        out_shape=jax.ShapeDtypeStruct((N, DH), jnp.float32),
    )(p, y, dis, Wh, bh)


def kernel(x, edge_index, W1, b1, W2, b2, W_age, b_age, W_sex, b_sex, W_eth, b_eth):
    pad = E_PAD - E
    src = jnp.concatenate([edge_index[0], jnp.zeros((pad,), jnp.int32)])
    src = src.reshape(NW, NCHUNK, CH)
    dst = jnp.concatenate([edge_index[1], jnp.full((pad,), TRASH, jnp.int32)])
    dst = dst.reshape(NW, NCHUNK, CH)
    idxp = jnp.stack([src, dst], axis=2)
    zeros128 = jnp.zeros((ROWS_PAD, D), jnp.float32)
    ones128 = jnp.ones((CH, D), jnp.float32)
    Wh = jnp.concatenate([W_age, W_sex, W_eth], axis=1)
    bh = jnp.concatenate([b_age, b_sex, b_eth]).reshape(1, -1)

    degp = _sc_degree(dst, zeros128, ones128)
    xw1 = _tc_matmul(x, W1)
    y1, dis = _tc_scale_from_deg(degp, xw1)

    p1 = _sc_aggregate(y1, idxp, zeros128)
    y2 = _tc_combine_matmul(p1, y1, dis, W2, b1.reshape(1, -1))

    p2 = _sc_aggregate(y2, idxp, zeros128)
    y3 = _tc_combine_scale(p2, y2, dis, b2.reshape(1, -1))

    p3 = _sc_aggregate(y3, idxp, zeros128)
    heads = _tc_heads(p3, y3, dis, Wh, bh)

    return (heads[:, 0:3], heads[:, 3:5], heads[:, 5:8])


# speedup vs baseline: 16.9963x; 1.1104x over previous
import functools

import jax
import jax.numpy as jnp
from jax import lax
from jax.experimental import pallas as pl
from jax.experimental.pallas import tpu as pltpu
from jax.experimental.pallas import tpu_sc as plsc

N = 10000
E = 320000
D = 128
NC = 2
NS = 16
NW = NC * NS
ROWS_PAD = 10240
RPS = ROWS_PAD // NS
CH = 128
NCHUNK = 79
E_PAD = NW * NCHUNK * CH
TRASH = ROWS_PAD - 1
NCH0 = 110
NCH1 = 48
C1BASE = NS * NCH0
BR = 2000


def _sc_mesh():
    return plsc.VectorSubcoreMesh(core_axis_name="c", subcore_axis_name="s")


def _sc_degree(dst, zeros, ones):
    @functools.partial(
        pl.kernel,
        mesh=_sc_mesh(),
        out_type=jax.ShapeDtypeStruct((NC, ROWS_PAD, D), jnp.float32),
        scratch_types=[
            pltpu.VMEM((NCHUNK, CH), jnp.int32),
            pltpu.VMEM((CH, D), jnp.float32),
            pltpu.VMEM_SHARED((ROWS_PAD, D), jnp.float32),
            pltpu.SemaphoreType.DMA,
            pltpu.SemaphoreType.DMA,
        ],
    )
    def k(dst_hbm, z_hbm, ones_hbm, out_hbm, di, ones_v, shared, s0, s1):
        cid = lax.axis_index("c")
        sid = lax.axis_index("s")
        wid = cid * NS + sid
        rows0 = sid * RPS
        pltpu.sync_copy(z_hbm.at[pl.ds(rows0, RPS)], shared.at[pl.ds(rows0, RPS)])
        pltpu.sync_copy(dst_hbm.at[wid], di)
        pltpu.sync_copy(ones_hbm, ones_v)
        plsc.subcore_barrier()

        @pl.loop(0, NCHUNK, step=2)
        def _(c):
            pltpu.async_copy(ones_v, shared.at[di.at[c]], s0, add=True)

            @pl.when(c + 1 < NCHUNK)
            def _():
                pltpu.async_copy(ones_v, shared.at[di.at[c + 1]], s1, add=True)

            pltpu.make_async_copy(ones_v, shared.at[di.at[c]], s0).wait()

            @pl.when(c + 1 < NCHUNK)
            def _():
                pltpu.make_async_copy(ones_v, shared.at[di.at[c + 1]], s1).wait()

        plsc.subcore_barrier()
        pltpu.sync_copy(shared.at[pl.ds(rows0, RPS)],
                        out_hbm.at[cid, pl.ds(rows0, RPS)])

    return k(dst, zeros, ones)


def _sc_aggregate(y, idx, zeros):
    @functools.partial(
        pl.kernel,
        mesh=_sc_mesh(),
        out_type=jax.ShapeDtypeStruct((NC, ROWS_PAD, D), jnp.float32),
        scratch_types=[
            pltpu.VMEM((2, CH), jnp.int32),
            pltpu.VMEM((2, CH), jnp.int32),
            pltpu.VMEM((CH, D), jnp.float32),
            pltpu.VMEM((CH, D), jnp.float32),
            pltpu.VMEM_SHARED((ROWS_PAD, D), jnp.float32),
            pltpu.SemaphoreType.DMA,
            pltpu.SemaphoreType.DMA,
            pltpu.SemaphoreType.DMA,
            pltpu.SemaphoreType.DMA,
        ],
    )
    def k(y_hbm, idx_hbm, z_hbm, out_hbm,
          ix0, ix1, r0, r1, shared, i0, i1, g0, g1):
        cid = lax.axis_index("c")
        sid = lax.axis_index("s")
        rows0 = sid * RPS
        start = jnp.where(cid == 0, sid * NCH0, C1BASE + sid * NCH1)
        nch = jnp.where(cid == 0, NCH0, NCH1)
        pltpu.sync_copy(z_hbm.at[pl.ds(rows0, RPS)], shared.at[pl.ds(rows0, RPS)])
        pltpu.sync_copy(idx_hbm.at[start], ix0)
        plsc.subcore_barrier()
        pltpu.async_copy(idx_hbm.at[start + 1], ix1, i1)
        pltpu.async_copy(y_hbm.at[ix0.at[0]], r0, g0)

        @pl.loop(0, NCH0, step=2)
        def _(c):
            @pl.when(c < nch)
            def _():
                @pl.when(c + 1 < nch)
                def _():
                    pltpu.make_async_copy(idx_hbm.at[start + c + 1], ix1, i1).wait()

                pltpu.make_async_copy(y_hbm.at[ix0.at[0]], r0, g0).wait()

                @pl.when(c + 1 < nch)
                def _():
                    pltpu.async_copy(y_hbm.at[ix1.at[0]], r1, g1)

                pltpu.sync_copy(r0, shared.at[ix0.at[1]], add=True)

                @pl.when(c + 2 < nch)
                def _():
                    pltpu.async_copy(idx_hbm.at[start + c + 2], ix0, i0)

                @pl.when(c + 1 < nch)
                def _():
                    pltpu.make_async_copy(y_hbm.at[ix1.at[0]], r1, g1).wait()

                    @pl.when(c + 2 < nch)
                    def _():
                        pltpu.make_async_copy(idx_hbm.at[start + c + 2], ix0, i0).wait()
                        pltpu.async_copy(y_hbm.at[ix0.at[0]], r0, g0)

                    pltpu.sync_copy(r1, shared.at[ix1.at[1]], add=True)

                    @pl.when(c + 3 < nch)
                    def _():
                        pltpu.async_copy(idx_hbm.at[start + c + 3], ix1, i1)

        plsc.subcore_barrier()
        pltpu.sync_copy(shared.at[pl.ds(rows0, RPS)],
                        out_hbm.at[cid, pl.ds(rows0, RPS)])

    return k(y, idx, zeros)


def _tc_matmul(x, W):
    def body(x_ref, w_ref, o_ref):
        o_ref[...] = jnp.dot(x_ref[...].astype(jnp.bfloat16),
                             w_ref[...].astype(jnp.bfloat16),
                             preferred_element_type=jnp.float32)

    return pl.pallas_call(
        body,
        grid=(N // BR,),
        in_specs=[pl.BlockSpec((BR, D), lambda i: (i, 0)),
                  pl.BlockSpec((D, D), lambda i: (0, 0))],
        out_specs=pl.BlockSpec((BR, D), lambda i: (i, 0)),
        out_shape=jax.ShapeDtypeStruct((N, D), jnp.float32),
    )(x, W)


def _tc_scale_from_deg(degp, xw):
    def body(degp_ref, xw_ref, y_ref, dis_ref):
        deg = degp_ref[0, :, 0:1] + degp_ref[1, :, 0:1] + 1.0
        dis = lax.rsqrt(deg)
        y_ref[...] = xw_ref[...] * dis
        dis_ref[...] = jnp.broadcast_to(dis, (BR, D))

    return pl.pallas_call(
        body,
        grid=(N // BR,),
        in_specs=[pl.BlockSpec((NC, BR, D), lambda i: (0, i, 0)),
                  pl.BlockSpec((BR, D), lambda i: (i, 0))],
        out_specs=[pl.BlockSpec((BR, D), lambda i: (i, 0)),
                   pl.BlockSpec((BR, D), lambda i: (i, 0))],
        out_shape=[jax.ShapeDtypeStruct((N, D), jnp.float32),
                   jax.ShapeDtypeStruct((N, D), jnp.float32)],
    )(degp, xw)


def _tc_combine_matmul(p, y, dis, W, b):
    def body(p_ref, y_ref, dis_ref, w_ref, b_ref, o_ref):
        s = p_ref[0] + p_ref[1] + y_ref[...]
        h = jnp.maximum(dis_ref[...] * s + b_ref[...], 0.0)
        o_ref[...] = dis_ref[...] * jnp.dot(h.astype(jnp.bfloat16),
                                            w_ref[...].astype(jnp.bfloat16),
                                            preferred_element_type=jnp.float32)

    return pl.pallas_call(
        body,
        grid=(N // BR,),
        in_specs=[pl.BlockSpec((NC, BR, D), lambda i: (0, i, 0)),
                  pl.BlockSpec((BR, D), lambda i: (i, 0)),
                  pl.BlockSpec((BR, D), lambda i: (i, 0)),
                  pl.BlockSpec((D, D), lambda i: (0, 0)),
                  pl.BlockSpec((1, D), lambda i: (0, 0))],
        out_specs=pl.BlockSpec((BR, D), lambda i: (i, 0)),
        out_shape=jax.ShapeDtypeStruct((N, D), jnp.float32),
    )(p, y, dis, W, b)


def _tc_combine_scale(p, y, dis, b):
    def body(p_ref, y_ref, dis_ref, b_ref, o_ref):
        s = p_ref[0] + p_ref[1] + y_ref[...]
        h = jnp.maximum(dis_ref[...] * s + b_ref[...], 0.0)
        o_ref[...] = dis_ref[...] * h

    return pl.pallas_call(
        body,
        grid=(N // BR,),
        in_specs=[pl.BlockSpec((NC, BR, D), lambda i: (0, i, 0)),
                  pl.BlockSpec((BR, D), lambda i: (i, 0)),
                  pl.BlockSpec((BR, D), lambda i: (i, 0)),
                  pl.BlockSpec((1, D), lambda i: (0, 0))],
        out_specs=pl.BlockSpec((BR, D), lambda i: (i, 0)),
        out_shape=jax.ShapeDtypeStruct((N, D), jnp.float32),
    )(p, y, dis, b)


def _tc_heads(p, y, dis, Wh, bh):
    DH = Wh.shape[1]

    def body(p_ref, y_ref, dis_ref, w_ref, b_ref, o_ref):
        agg = dis_ref[...] * (p_ref[0] + p_ref[1] + y_ref[...])
        o_ref[...] = jnp.dot(agg.astype(jnp.bfloat16),
                             w_ref[...].astype(jnp.bfloat16),
                             preferred_element_type=jnp.float32) + b_ref[...]

    return pl.pallas_call(
        body,
        grid=(N // BR,),
        in_specs=[pl.BlockSpec((NC, BR, D), lambda i: (0, i, 0)),
                  pl.BlockSpec((BR, D), lambda i: (i, 0)),
                  pl.BlockSpec((BR, D), lambda i: (i, 0)),
                  pl.BlockSpec((D, DH), lambda i: (0, 0)),
                  pl.BlockSpec((1, DH), lambda i: (0, 0))],
        out_specs=pl.BlockSpec((BR, DH), lambda i: (i, 0)),
        out_shape=jax.ShapeDtypeStruct((N, DH), jnp.float32),
    )(p, y, dis, Wh, bh)


def kernel(x, edge_index, W1, b1, W2, b2, W_age, b_age, W_sex, b_sex, W_eth, b_eth):
    pad = E_PAD - E
    src = jnp.concatenate([edge_index[0], jnp.zeros((pad,), jnp.int32)])
    dst = jnp.concatenate([edge_index[1], jnp.full((pad,), TRASH, jnp.int32)])
    idxp = jnp.stack([src.reshape(NW * NCHUNK, CH),
                      dst.reshape(NW * NCHUNK, CH)], axis=1)
    dst3 = dst.reshape(NW, NCHUNK, CH)
    zeros128 = jnp.zeros((ROWS_PAD, D), jnp.float32)
    ones128 = jnp.ones((CH, D), jnp.float32)
    Wh = jnp.concatenate([W_age, W_sex, W_eth], axis=1)
    bh = jnp.concatenate([b_age, b_sex, b_eth]).reshape(1, -1)

    degp = _sc_degree(dst3, zeros128, ones128)
    xw1 = _tc_matmul(x, W1)
    y1, dis = _tc_scale_from_deg(degp, xw1)

    p1 = _sc_aggregate(y1, idxp, zeros128)
    y2 = _tc_combine_matmul(p1, y1, dis, W2, b1.reshape(1, -1))

    p2 = _sc_aggregate(y2, idxp, zeros128)
    y3 = _tc_combine_scale(p2, y2, dis, b2.reshape(1, -1))

    p3 = _sc_aggregate(y3, idxp, zeros128)
    heads = _tc_heads(p3, y3, dis, Wh, bh)

    return (heads[:, 0:3], heads[:, 3:5], heads[:, 5:8])


# speedup vs baseline: 21.6376x; 1.2731x over previous
import functools

import jax
import jax.numpy as jnp
from jax import lax
from jax.experimental import pallas as pl
from jax.experimental.pallas import tpu as pltpu
from jax.experimental.pallas import tpu_sc as plsc

N = 10000
E = 320000
D = 128
NC = 2
NS = 16
NW = NC * NS
ROWS_PAD = 10112
RPS = ROWS_PAD // NS
CH = 120
NCHUNK = 84
E_PAD = NW * NCHUNK * CH
TRASH = ROWS_PAD - 1
NCH0 = 120
NCH1 = 48
C1BASE = NS * NCH0
BR = 2000


def _sc_mesh():
    return plsc.VectorSubcoreMesh(core_axis_name="c", subcore_axis_name="s")


def _sc_degree(dst, zeros, ones):
    @functools.partial(
        pl.kernel,
        mesh=_sc_mesh(),
        out_type=jax.ShapeDtypeStruct((NC, ROWS_PAD, D), jnp.float32),
        scratch_types=[
            pltpu.VMEM((NCHUNK, CH), jnp.int32),
            pltpu.VMEM((CH, D), jnp.float32),
            pltpu.VMEM_SHARED((ROWS_PAD, D), jnp.float32),
            pltpu.SemaphoreType.DMA,
            pltpu.SemaphoreType.DMA,
        ],
    )
    def k(dst_hbm, z_hbm, ones_hbm, out_hbm, di, ones_v, shared, s0, s1):
        cid = lax.axis_index("c")
        sid = lax.axis_index("s")
        wid = cid * NS + sid
        rows0 = sid * RPS
        pltpu.sync_copy(z_hbm.at[pl.ds(rows0, RPS)], shared.at[pl.ds(rows0, RPS)])
        pltpu.sync_copy(dst_hbm.at[wid], di)
        pltpu.sync_copy(ones_hbm, ones_v)
        plsc.subcore_barrier()

        @pl.loop(0, NCHUNK, step=2)
        def _(c):
            pltpu.async_copy(ones_v, shared.at[di.at[c]], s0, add=True)

            @pl.when(c + 1 < NCHUNK)
            def _():
                pltpu.async_copy(ones_v, shared.at[di.at[c + 1]], s1, add=True)

            pltpu.make_async_copy(ones_v, shared.at[di.at[c]], s0).wait()

            @pl.when(c + 1 < NCHUNK)
            def _():
                pltpu.make_async_copy(ones_v, shared.at[di.at[c + 1]], s1).wait()

        plsc.subcore_barrier()
        pltpu.sync_copy(shared.at[pl.ds(rows0, RPS)],
                        out_hbm.at[cid, pl.ds(rows0, RPS)])

    return k(dst, zeros, ones)


def _sc_aggregate(y, idx, zeros):
    @functools.partial(
        pl.kernel,
        mesh=_sc_mesh(),
        out_type=jax.ShapeDtypeStruct((NC, ROWS_PAD, D), jnp.float32),
        scratch_types=(
            [pltpu.VMEM((2, CH), jnp.int32)] * 6
            + [pltpu.VMEM((CH, D), jnp.float32)] * 3
            + [pltpu.VMEM_SHARED((ROWS_PAD, D), jnp.float32)]
            + [pltpu.SemaphoreType.DMA] * 12
        ),
    )
    def k(y_hbm, idx_hbm, z_hbm, out_hbm, *scr):
        ixs = scr[0:6]
        rs = scr[6:9]
        shared = scr[9]
        isem = scr[10:16]
        gsem = scr[16:19]
        ssem = scr[19:22]
        cid = lax.axis_index("c")
        sid = lax.axis_index("s")
        rows0 = sid * RPS
        start = jnp.where(cid == 0, sid * NCH0, C1BASE + sid * NCH1)
        nch = jnp.where(cid == 0, NCH0, NCH1)
        pltpu.sync_copy(z_hbm.at[pl.ds(rows0, RPS)], shared.at[pl.ds(rows0, RPS)])
        pltpu.sync_copy(idx_hbm.at[start], ixs[0])
        plsc.subcore_barrier()
        for j in range(1, 5):
            pltpu.async_copy(idx_hbm.at[start + j], ixs[j], isem[j])
        pltpu.async_copy(y_hbm.at[ixs[0].at[0]], rs[0], gsem[0])
        pltpu.make_async_copy(idx_hbm.at[start + 1], ixs[1], isem[1]).wait()
        pltpu.async_copy(y_hbm.at[ixs[1].at[0]], rs[1], gsem[1])

        @pl.loop(0, NCH0, step=6)
        def _(t):
            for j in range(6):
                b = j % 3

                @pl.when(t + j < nch)
                def _(u=t + j, j=j, b=b):
                    pltpu.make_async_copy(y_hbm.at[ixs[j].at[0]],
                                          rs[b], gsem[b]).wait()
                    pltpu.async_copy(rs[b], shared.at[ixs[j].at[1]],
                                     ssem[b], add=True)

                    @pl.when(u >= 1)
                    def _():
                        pltpu.make_async_copy(rs[(b + 2) % 3],
                                              shared.at[ixs[(j + 5) % 6].at[1]],
                                              ssem[(b + 2) % 3]).wait()

                    @pl.when(u + 2 < nch)
                    def _():
                        pltpu.make_async_copy(idx_hbm.at[start + u + 2],
                                              ixs[(j + 2) % 6],
                                              isem[(j + 2) % 6]).wait()
                        pltpu.async_copy(y_hbm.at[ixs[(j + 2) % 6].at[0]],
                                         rs[(b + 2) % 3], gsem[(b + 2) % 3])

                    @pl.when(u + 5 < nch)
                    def _():
                        pltpu.async_copy(idx_hbm.at[start + u + 5],
                                         ixs[(j + 5) % 6], isem[(j + 5) % 6])

        pltpu.make_async_copy(rs[2], shared.at[ixs[(NCH1 - 1) % 6].at[1]],
                              ssem[2]).wait()
        plsc.subcore_barrier()
        pltpu.sync_copy(shared.at[pl.ds(rows0, RPS)],
                        out_hbm.at[cid, pl.ds(rows0, RPS)])

    return k(y, idx, zeros)


def _tc_matmul(x, W):
    def body(x_ref, w_ref, o_ref):
        o_ref[...] = jnp.dot(x_ref[...].astype(jnp.bfloat16),
                             w_ref[...].astype(jnp.bfloat16),
                             preferred_element_type=jnp.float32)

    return pl.pallas_call(
        body,
        grid=(N // BR,),
        in_specs=[pl.BlockSpec((BR, D), lambda i: (i, 0)),
                  pl.BlockSpec((D, D), lambda i: (0, 0))],
        out_specs=pl.BlockSpec((BR, D), lambda i: (i, 0)),
        out_shape=jax.ShapeDtypeStruct((N, D), jnp.float32),
    )(x, W)


def _tc_scale_from_deg(degp, xw):
    def body(degp_ref, xw_ref, y_ref, dis_ref):
        deg = degp_ref[0, :, 0:1] + degp_ref[1, :, 0:1] + 1.0
        dis = lax.rsqrt(deg)
        y_ref[...] = xw_ref[...] * dis
        dis_ref[...] = jnp.broadcast_to(dis, (BR, D))

    return pl.pallas_call(
        body,
        grid=(N // BR,),
        in_specs=[pl.BlockSpec((NC, BR, D), lambda i: (0, i, 0)),
                  pl.BlockSpec((BR, D), lambda i: (i, 0))],
        out_specs=[pl.BlockSpec((BR, D), lambda i: (i, 0)),
                   pl.BlockSpec((BR, D), lambda i: (i, 0))],
        out_shape=[jax.ShapeDtypeStruct((N, D), jnp.float32),
                   jax.ShapeDtypeStruct((N, D), jnp.float32)],
    )(degp, xw)


def _tc_combine_matmul(p, y, dis, W, b):
    def body(p_ref, y_ref, dis_ref, w_ref, b_ref, o_ref):
        s = p_ref[0] + p_ref[1] + y_ref[...]
        h = jnp.maximum(dis_ref[...] * s + b_ref[...], 0.0)
        o_ref[...] = dis_ref[...] * jnp.dot(h.astype(jnp.bfloat16),
                                            w_ref[...].astype(jnp.bfloat16),
                                            preferred_element_type=jnp.float32)

    return pl.pallas_call(
        body,
        grid=(N // BR,),
        in_specs=[pl.BlockSpec((NC, BR, D), lambda i: (0, i, 0)),
                  pl.BlockSpec((BR, D), lambda i: (i, 0)),
                  pl.BlockSpec((BR, D), lambda i: (i, 0)),
                  pl.BlockSpec((D, D), lambda i: (0, 0)),
                  pl.BlockSpec((1, D), lambda i: (0, 0))],
        out_specs=pl.BlockSpec((BR, D), lambda i: (i, 0)),
        out_shape=jax.ShapeDtypeStruct((N, D), jnp.float32),
    )(p, y, dis, W, b)


def _tc_combine_scale(p, y, dis, b):
    def body(p_ref, y_ref, dis_ref, b_ref, o_ref):
        s = p_ref[0] + p_ref[1] + y_ref[...]
        h = jnp.maximum(dis_ref[...] * s + b_ref[...], 0.0)
        o_ref[...] = dis_ref[...] * h

    return pl.pallas_call(
        body,
        grid=(N // BR,),
        in_specs=[pl.BlockSpec((NC, BR, D), lambda i: (0, i, 0)),
                  pl.BlockSpec((BR, D), lambda i: (i, 0)),
                  pl.BlockSpec((BR, D), lambda i: (i, 0)),
                  pl.BlockSpec((1, D), lambda i: (0, 0))],
        out_specs=pl.BlockSpec((BR, D), lambda i: (i, 0)),
        out_shape=jax.ShapeDtypeStruct((N, D), jnp.float32),
    )(p, y, dis, b)


def _tc_heads(p, y, dis, Wh, bh):
    DH = Wh.shape[1]

    def body(p_ref, y_ref, dis_ref, w_ref, b_ref, o_ref):
        agg = dis_ref[...] * (p_ref[0] + p_ref[1] + y_ref[...])
        o_ref[...] = jnp.dot(agg.astype(jnp.bfloat16),
                             w_ref[...].astype(jnp.bfloat16),
                             preferred_element_type=jnp.float32) + b_ref[...]

    return pl.pallas_call(
        body,
        grid=(N // BR,),
        in_specs=[pl.BlockSpec((NC, BR, D), lambda i: (0, i, 0)),
                  pl.BlockSpec((BR, D), lambda i: (i, 0)),
                  pl.BlockSpec((BR, D), lambda i: (i, 0)),
                  pl.BlockSpec((D, DH), lambda i: (0, 0)),
                  pl.BlockSpec((1, DH), lambda i: (0, 0))],
        out_specs=pl.BlockSpec((BR, DH), lambda i: (i, 0)),
        out_shape=jax.ShapeDtypeStruct((N, DH), jnp.float32),
    )(p, y, dis, Wh, bh)


def kernel(x, edge_index, W1, b1, W2, b2, W_age, b_age, W_sex, b_sex, W_eth, b_eth):
    pad = E_PAD - E
    src = jnp.concatenate([edge_index[0], jnp.zeros((pad,), jnp.int32)])
    dst = jnp.concatenate([edge_index[1], jnp.full((pad,), TRASH, jnp.int32)])
    idxp = jnp.stack([src.reshape(NW * NCHUNK, CH),
                      dst.reshape(NW * NCHUNK, CH)], axis=1)
    dst3 = dst.reshape(NW, NCHUNK, CH)
    zeros128 = jnp.zeros((ROWS_PAD, D), jnp.float32)
    ones128 = jnp.ones((CH, D), jnp.float32)
    Wh = jnp.concatenate([W_age, W_sex, W_eth], axis=1)
    bh = jnp.concatenate([b_age, b_sex, b_eth]).reshape(1, -1)

    degp = _sc_degree(dst3, zeros128, ones128)
    xw1 = _tc_matmul(x, W1)
    y1, dis = _tc_scale_from_deg(degp, xw1)

    p1 = _sc_aggregate(y1, idxp, zeros128)
    y2 = _tc_combine_matmul(p1, y1, dis, W2, b1.reshape(1, -1))

    p2 = _sc_aggregate(y2, idxp, zeros128)
    y3 = _tc_combine_scale(p2, y2, dis, b2.reshape(1, -1))

    p3 = _sc_aggregate(y3, idxp, zeros128)
    heads = _tc_heads(p3, y3, dis, Wh, bh)

    return (heads[:, 0:3], heads[:, 3:5], heads[:, 5:8])
